# baseline (speedup 1.0000x reference)
import jax
import jax.numpy as jnp
from jax import lax
from jax.experimental import pallas as pl
from jax.experimental.pallas import tpu as pltpu
from jax.experimental.pallas import tpu_sc as plsc

NFEAT = 26
BATCH = 16384
DIM = 32
VOCAB = 100000
NC = 2
NS = 16
NW = NC * NS
BPW = BATCH // NW
NBUF = 4
GCH = 128
GCH_N = BPW // GCH


def _gather_body(*refs):
    idx_refs = refs[:NFEAT]
    tab_refs = refs[NFEAT:2 * NFEAT]
    out_ref = refs[2 * NFEAT]
    rest = refs[2 * NFEAT + 1:]
    idx_all = rest[0]
    bufs = rest[1:1 + NBUF]
    gsems = rest[1 + NBUF:1 + 2 * NBUF]
    wsems = rest[1 + 2 * NBUF:1 + 3 * NBUF]
    isem = rest[1 + 3 * NBUF]
    wid = lax.axis_index("s") * NC + lax.axis_index("c")
    base = wid * BPW

    ih = [pltpu.async_copy(idx_refs[f].at[pl.ds(base, BPW)], idx_all.at[f], isem)
          for f in range(NFEAT)]
    for h in ih:
        h.wait()

    steps = [(f, k) for f in range(NFEAT) for k in range(GCH_N)]
    nst = len(steps)

    def gather_start(i, s):
        f, k = steps[i]
        return pltpu.async_copy(
            tab_refs[f].at[idx_all.at[f, pl.ds(k * GCH, GCH)]],
            bufs[s], gsems[s])

    def write_start(i, s):
        f, k = steps[i]
        return pltpu.async_copy(
            bufs[s],
            out_ref.at[pl.ds(base + k * GCH, GCH), pl.ds(f * DIM, DIM)],
            wsems[s])

    hg = [None] * NBUF
    hw = [None] * NBUF
    for i in range(nst):
        s = i % NBUF
        if i >= NBUF:
            hw[s].wait()
        hg[s] = gather_start(i, s)
        if i >= NBUF - 1:
            ip = i - (NBUF - 1)
            sp = ip % NBUF
            hg[sp].wait()
            hw[sp] = write_start(ip, sp)
    for ip in range(nst - (NBUF - 1), nst):
        sp = ip % NBUF
        hg[sp].wait()
        hw[sp] = write_start(ip, sp)
    for sp in set(ip % NBUF for ip in range(nst - NBUF, nst)):
        hw[sp].wait()


def kernel(f00, f01, f02, f03, f04, f05, f06, f07, f08, f09, f10, f11, f12, f13, f14, f15, f16, f17, f18, f19, f20, f21, f22, f23, f24, f25, W_f00, W_f01, W_f02, W_f03, W_f04, W_f05, W_f06, W_f07, W_f08, W_f09, W_f10, W_f11, W_f12, W_f13, W_f14, W_f15, W_f16, W_f17, W_f18, W_f19, W_f20, W_f21, W_f22, W_f23, W_f24, W_f25):
    raw_idx = (f00, f01, f02, f03, f04, f05, f06, f07, f08, f09, f10, f11,
               f12, f13, f14, f15, f16, f17, f18, f19, f20, f21, f22, f23,
               f24, f25)
    idxs = [jnp.asarray(x, jnp.int32) for x in raw_idx]
    tabs = [W_f00, W_f01, W_f02, W_f03, W_f04, W_f05, W_f06, W_f07, W_f08,
            W_f09, W_f10, W_f11, W_f12, W_f13, W_f14, W_f15, W_f16, W_f17,
            W_f18, W_f19, W_f20, W_f21, W_f22, W_f23, W_f24, W_f25]
    mesh = plsc.VectorSubcoreMesh(
        core_axis_name="c", subcore_axis_name="s", num_cores=NC, num_subcores=NS)
    run = pl.kernel(
        _gather_body,
        out_type=jax.ShapeDtypeStruct((BATCH, NFEAT * DIM), jnp.float32),
        mesh=mesh,
        compiler_params=pltpu.CompilerParams(use_tc_tiling_on_sc=False),
        scratch_types=(
            [pltpu.VMEM((NFEAT, BPW), jnp.int32)]
            + [pltpu.VMEM((GCH, DIM), jnp.float32) for _ in range(NBUF)]
            + [pltpu.SemaphoreType.DMA for _ in range(2 * NBUF + 1)]
        ),
    )
    return run(*idxs, *tabs)
